# baseline (speedup 1.0000x reference)
import functools
import jax
import jax.numpy as jnp
from jax import lax
from jax.experimental import pallas as pl
from jax.experimental.pallas import tpu as pltpu
from jax.experimental.pallas import tpu_sc as plsc

B, H, W, C = 4, 512, 512, 96
NW = 32
PER_W = B * H * 2 * 2 // NW


@functools.cache
def _make_probe():
    mesh = plsc.VectorSubcoreMesh(core_axis_name="c", subcore_axis_name="s")
    return functools.partial(
        pl.kernel,
        out_type=jax.ShapeDtypeStruct((NW, 16), jnp.float32),
        mesh=mesh,
        compiler_params=pltpu.CompilerParams(needs_layout_passes=False),
        scratch_types=[
            pltpu.VMEM((2, W // 2, C), jnp.float32),
            pltpu.VMEM((16,), jnp.float32),
            pltpu.SemaphoreType.DMA((2,)),
        ],
    )(_probe_body)


def _probe_body(pred_hbm, true_hbm, out_hbm, buf, res_v, sems):
    cid = lax.axis_index("c")
    sid = lax.axis_index("s")
    wid = sid * 2 + cid
    base = wid * PER_W

    def issue(rs, slot):
        a = lax.shift_right_logical(rs, 12)
        b = lax.shift_right_logical(rs, 10) & 3
        r = lax.shift_right_logical(rs, 1) & 511
        w0 = (rs & 1) * (W // 2)

        @pl.when(a == 0)
        def _():
            pltpu.make_async_copy(pred_hbm.at[b, r, pl.ds(w0, W // 2)],
                                  buf.at[slot], sems.at[slot]).start()

        @pl.when(a != 0)
        def _():
            pltpu.make_async_copy(true_hbm.at[b, r, pl.ds(w0, W // 2)],
                                  buf.at[slot], sems.at[slot]).start()

    issue(base, jnp.int32(0))

    def body(j, acc):
        slot = j & 1

        @pl.when(j + 1 < PER_W)
        def _():
            issue(base + j + 1, (j + 1) & 1)

        pltpu.make_async_copy(pred_hbm.at[0, 0, pl.ds(0, W // 2)], buf.at[slot], sems.at[slot]).wait()
        return jnp.maximum(acc, buf[slot, 0, pl.ds(0, 16)])

    acc = lax.fori_loop(0, PER_W, body, jnp.full((16,), -1e30, jnp.float32))
    res_v[...] = acc
    pltpu.make_async_copy(res_v, out_hbm.at[wid], sems.at[0]).start()
    pltpu.make_async_copy(res_v, out_hbm.at[wid], sems.at[0]).wait()


BH = 16
NH = H // BH


def _tc_probe_kernel(pred_ref, true_ref, o1, o2):
    h = pl.program_id(1)
    first = h == 0

    def one(ref, o):
        x = ref[0]
        m = jnp.max(jnp.max(x, axis=0), axis=-1)

        @pl.when(first)
        def _():
            o[0, 0, :] = m

        @pl.when(jnp.logical_not(first))
        def _():
            o[0, 0, :] = jnp.maximum(o[0, 0, :], m)

    one(pred_ref, o1)
    one(true_ref, o2)


def _tc_probe(pred, true):
    out_sds = jax.ShapeDtypeStruct((B, 1, W), jnp.float32)
    in_spec = pl.BlockSpec((1, BH, W, C), lambda b, h: (b, h, 0, 0))
    out_spec = pl.BlockSpec((1, 1, W), lambda b, h: (b, 0, 0))
    return pl.pallas_call(
        _tc_probe_kernel,
        grid=(B, NH),
        in_specs=[in_spec, in_spec],
        out_specs=[out_spec] * 2,
        out_shape=[out_sds] * 2,
        compiler_params=pltpu.CompilerParams(
            dimension_semantics=("parallel", "arbitrary")),
    )(pred, true)


def kernel(prediction_probs, expected_onehot):
    o = _make_probe()(prediction_probs, expected_onehot)
    t1, t2 = _tc_probe(prediction_probs, expected_onehot)
    return 0.05 * jnp.mean(o[:, 0]) + 1e-9 * (jnp.mean(t1) + jnp.mean(t2))


# speedup vs baseline: 5.9397x; 5.9397x over previous
import functools

import jax
import jax.numpy as jnp
from jax import lax
from jax.experimental import pallas as pl
from jax.experimental.pallas import tpu as pltpu
from jax.experimental.pallas import tpu_sc as plsc

B, H, W, C = 4, 512, 512, 96
BH = 16
NH = H // BH
L = 16


def _bbox_stage_kernel(pred_ref, true_ref, p_ymin_ref, p_ymax_ref,
                       t_ymin_ref, t_ymax_ref):
    h = pl.program_id(1)
    first = h == 0
    row_f = (h * BH).astype(jnp.float32) + lax.broadcasted_iota(
        jnp.int32, (BH, W), 0).astype(jnp.float32)

    def one_side(x_ref, ymin_ref, ymax_ref):
        x = x_ref[0]
        mask = jnp.max(x, axis=1) > x[:, 0, :]
        ymin_p = jnp.min(jnp.where(mask, row_f, float(H)), axis=0)
        ymax_p = jnp.max(jnp.where(mask, row_f, -1.0), axis=0)

        @pl.when(first)
        def _():
            ymin_ref[0, 0, :] = ymin_p
            ymax_ref[0, 0, :] = ymax_p

        @pl.when(jnp.logical_not(first))
        def _():
            ymin_ref[0, 0, :] = jnp.minimum(ymin_ref[0, 0, :], ymin_p)
            ymax_ref[0, 0, :] = jnp.maximum(ymax_ref[0, 0, :], ymax_p)

    one_side(pred_ref, p_ymin_ref, p_ymax_ref)
    one_side(true_ref, t_ymin_ref, t_ymax_ref)


def _bbox_vectors(pred_t, true_t):
    out_sds = jax.ShapeDtypeStruct((B, 1, W), jnp.float32)
    in_spec = pl.BlockSpec((1, BH, C, W), lambda b, h: (b, h, 0, 0))
    out_spec = pl.BlockSpec((1, 1, W), lambda b, h: (b, 0, 0))
    return pl.pallas_call(
        _bbox_stage_kernel,
        grid=(B, NH),
        in_specs=[in_spec, in_spec],
        out_specs=[out_spec] * 4,
        out_shape=[out_sds] * 4,
        compiler_params=pltpu.CompilerParams(
            dimension_semantics=("parallel", "arbitrary")),
    )(pred_t, true_t)


def _splat(x):
    return jnp.full((L,), x, jnp.float32)


def _sqrt16(d):
    bits = lax.bitcast_convert_type(d, jnp.int32)
    y = lax.bitcast_convert_type(
        jnp.int32(0x1FBD1DF5) + lax.shift_right_logical(bits, 1), jnp.float32)
    for _ in range(4):
        y = 0.5 * (y + d / y)
    return y


def _tanh16(x):
    return 1.0 - 2.0 / (jnp.exp(2.0 * x) + 1.0)


def _side_box(ymin_ref, ymax_ref):
    init = (_splat(float(H)), _splat(-1.0), _splat(float(W)), _splat(-1.0))

    def body(i, carry):
        a_ymn, a_ymx, a_xmn, a_xmx = carry
        off = i * L
        yn = ymin_ref[pl.ds(off, L)]
        yx = ymax_ref[pl.ds(off, L)]
        wf = (lax.iota(jnp.int32, L) + off).astype(jnp.float32)
        active = yx >= 0.0
        return (jnp.minimum(a_ymn, yn),
                jnp.maximum(a_ymx, yx),
                jnp.minimum(a_xmn, jnp.where(active, wf, float(W))),
                jnp.maximum(a_xmx, jnp.where(active, wf, -1.0)))

    a_ymn, a_ymx, a_xmn, a_xmx = lax.fori_loop(0, W // L, body, init)
    ymn = _splat(jnp.min(a_ymn))
    ymx = _splat(jnp.max(a_ymx))
    xmn = _splat(jnp.min(a_xmn))
    xmx = _splat(jnp.max(a_xmx))
    nonempty = ymx >= 0.0
    y0 = jnp.where(nonempty, ymn, 0.0)
    x0 = jnp.where(nonempty, xmn, 0.0)
    y1 = jnp.where(nonempty, ymx, 1.0)
    x1 = jnp.where(nonempty, xmx, 1.0)
    return y0, x0, y1, x1, nonempty


@functools.cache
def _make_penalty_kernel():
    mesh = plsc.VectorSubcoreMesh(core_axis_name="c", subcore_axis_name="s")
    return functools.partial(
        pl.kernel,
        out_type=jax.ShapeDtypeStruct((B, L), jnp.float32),
        mesh=mesh,
        compiler_params=pltpu.CompilerParams(needs_layout_passes=False),
        scratch_types=[
            pltpu.VMEM((W,), jnp.float32),
            pltpu.VMEM((W,), jnp.float32),
            pltpu.VMEM((W,), jnp.float32),
            pltpu.VMEM((W,), jnp.float32),
            pltpu.VMEM((L,), jnp.float32),
        ],
    )(_penalty_body)


def _penalty_body(p_ymin_hbm, p_ymax_hbm, t_ymin_hbm, t_ymax_hbm, out_hbm,
                  pn_v, px_v, tn_v, tx_v, res_v):
    cid = lax.axis_index("c")
    sid = lax.axis_index("s")

    @pl.when((cid == 0) & (sid < B))
    def _():
        b = sid
        pltpu.sync_copy(p_ymin_hbm.at[b, 0], pn_v)
        pltpu.sync_copy(p_ymax_hbm.at[b, 0], px_v)
        pltpu.sync_copy(t_ymin_hbm.at[b, 0], tn_v)
        pltpu.sync_copy(t_ymax_hbm.at[b, 0], tx_v)
        py0, px0, py1, px1, p_any = _side_box(pn_v, px_v)
        ty0, tx0, ty1, tx1, t_any = _side_box(tn_v, tx_v)

        pred_area = (py1 - py0 + 1.0) * (px1 - px0 + 1.0)
        true_area = (ty1 - ty0 + 1.0) * (tx1 - tx0 + 1.0)
        area_pen = jnp.maximum(pred_area - true_area, 0.0) / (true_area + 1.0)
        cdy = (py0 + py1) / 2.0 - (ty0 + ty1) / 2.0
        cdx = (px0 + px1) / 2.0 - (tx0 + tx1) / 2.0
        center_pen = _sqrt16(cdy * cdy + cdx * cdx) / 20.0
        iy0 = jnp.maximum(py0, ty0)
        ix0 = jnp.maximum(px0, tx0)
        iy1 = jnp.minimum(py1, ty1)
        ix1 = jnp.minimum(px1, tx1)
        inter = (jnp.maximum(0.0, iy1 - iy0 + 1.0)
                 * jnp.maximum(0.0, ix1 - ix0 + 1.0))
        union = pred_area + true_area - inter + 1e-6
        total = area_pen + center_pen + 1.0 - inter / union
        pen = jnp.where(p_any & t_any, _tanh16(total), _splat(0.0))
        res_v[...] = pen
        pltpu.sync_copy(res_v, out_hbm.at[b])


def kernel(prediction_probs, expected_onehot):
    pred_t = jnp.transpose(prediction_probs, (0, 1, 3, 2))
    true_t = jnp.transpose(expected_onehot, (0, 1, 3, 2))
    p_ymin, p_ymax, t_ymin, t_ymax = _bbox_vectors(pred_t, true_t)
    pens = _make_penalty_kernel()(p_ymin, p_ymax, t_ymin, t_ymax)
    return 0.05 * jnp.mean(pens[:, 0])


# speedup vs baseline: 6.0233x; 1.0141x over previous
import functools

import jax
import jax.numpy as jnp
from jax import lax
from jax.experimental import pallas as pl
from jax.experimental.pallas import tpu as pltpu
from jax.experimental.pallas import tpu_sc as plsc

B, H, W, C = 4, 512, 512, 96
BH = 32
NH = H // BH
L = 16


def _bbox_stage_kernel(pred_ref, true_ref, p_ymin_ref, p_ymax_ref,
                       t_ymin_ref, t_ymax_ref):
    h = pl.program_id(1)
    first = h == 0
    row_f = (h * BH).astype(jnp.float32) + lax.broadcasted_iota(
        jnp.int32, (BH, W), 0).astype(jnp.float32)

    def one_side(x_ref, ymin_ref, ymax_ref):
        x = x_ref[0]
        mask = jnp.max(x, axis=1) > x[:, 0, :]
        ymin_p = jnp.min(jnp.where(mask, row_f, float(H)), axis=0)
        ymax_p = jnp.max(jnp.where(mask, row_f, -1.0), axis=0)

        @pl.when(first)
        def _():
            ymin_ref[0, 0, :] = ymin_p
            ymax_ref[0, 0, :] = ymax_p

        @pl.when(jnp.logical_not(first))
        def _():
            ymin_ref[0, 0, :] = jnp.minimum(ymin_ref[0, 0, :], ymin_p)
            ymax_ref[0, 0, :] = jnp.maximum(ymax_ref[0, 0, :], ymax_p)

    one_side(pred_ref, p_ymin_ref, p_ymax_ref)
    one_side(true_ref, t_ymin_ref, t_ymax_ref)


def _bbox_vectors(pred_t, true_t):
    out_sds = jax.ShapeDtypeStruct((B, 1, W), jnp.float32)
    in_spec = pl.BlockSpec((1, BH, C, W), lambda b, h: (b, h, 0, 0))
    out_spec = pl.BlockSpec((1, 1, W), lambda b, h: (b, 0, 0))
    return pl.pallas_call(
        _bbox_stage_kernel,
        grid=(B, NH),
        in_specs=[in_spec, in_spec],
        out_specs=[out_spec] * 4,
        out_shape=[out_sds] * 4,
        compiler_params=pltpu.CompilerParams(
            dimension_semantics=("parallel", "arbitrary")),
    )(pred_t, true_t)


def _splat(x):
    return jnp.full((L,), x, jnp.float32)


def _sqrt16(d):
    bits = lax.bitcast_convert_type(d, jnp.int32)
    y = lax.bitcast_convert_type(
        jnp.int32(0x1FBD1DF5) + lax.shift_right_logical(bits, 1), jnp.float32)
    for _ in range(4):
        y = 0.5 * (y + d / y)
    return y


def _tanh16(x):
    return 1.0 - 2.0 / (jnp.exp(2.0 * x) + 1.0)


def _side_box(ymin_ref, ymax_ref):
    init = (_splat(float(H)), _splat(-1.0), _splat(float(W)), _splat(-1.0))

    def body(i, carry):
        a_ymn, a_ymx, a_xmn, a_xmx = carry
        off = i * L
        yn = ymin_ref[pl.ds(off, L)]
        yx = ymax_ref[pl.ds(off, L)]
        wf = (lax.iota(jnp.int32, L) + off).astype(jnp.float32)
        active = yx >= 0.0
        return (jnp.minimum(a_ymn, yn),
                jnp.maximum(a_ymx, yx),
                jnp.minimum(a_xmn, jnp.where(active, wf, float(W))),
                jnp.maximum(a_xmx, jnp.where(active, wf, -1.0)))

    a_ymn, a_ymx, a_xmn, a_xmx = lax.fori_loop(0, W // L, body, init)
    ymn = _splat(jnp.min(a_ymn))
    ymx = _splat(jnp.max(a_ymx))
    xmn = _splat(jnp.min(a_xmn))
    xmx = _splat(jnp.max(a_xmx))
    nonempty = ymx >= 0.0
    y0 = jnp.where(nonempty, ymn, 0.0)
    x0 = jnp.where(nonempty, xmn, 0.0)
    y1 = jnp.where(nonempty, ymx, 1.0)
    x1 = jnp.where(nonempty, xmx, 1.0)
    return y0, x0, y1, x1, nonempty


@functools.cache
def _make_penalty_kernel():
    mesh = plsc.VectorSubcoreMesh(core_axis_name="c", subcore_axis_name="s")
    return functools.partial(
        pl.kernel,
        out_type=jax.ShapeDtypeStruct((B, L), jnp.float32),
        mesh=mesh,
        compiler_params=pltpu.CompilerParams(needs_layout_passes=False),
        scratch_types=[
            pltpu.VMEM((W,), jnp.float32),
            pltpu.VMEM((W,), jnp.float32),
            pltpu.VMEM((W,), jnp.float32),
            pltpu.VMEM((W,), jnp.float32),
            pltpu.VMEM((L,), jnp.float32),
        ],
    )(_penalty_body)


def _penalty_body(p_ymin_hbm, p_ymax_hbm, t_ymin_hbm, t_ymax_hbm, out_hbm,
                  pn_v, px_v, tn_v, tx_v, res_v):
    cid = lax.axis_index("c")
    sid = lax.axis_index("s")

    @pl.when((cid == 0) & (sid < B))
    def _():
        b = sid
        pltpu.sync_copy(p_ymin_hbm.at[b, 0], pn_v)
        pltpu.sync_copy(p_ymax_hbm.at[b, 0], px_v)
        pltpu.sync_copy(t_ymin_hbm.at[b, 0], tn_v)
        pltpu.sync_copy(t_ymax_hbm.at[b, 0], tx_v)
        py0, px0, py1, px1, p_any = _side_box(pn_v, px_v)
        ty0, tx0, ty1, tx1, t_any = _side_box(tn_v, tx_v)

        pred_area = (py1 - py0 + 1.0) * (px1 - px0 + 1.0)
        true_area = (ty1 - ty0 + 1.0) * (tx1 - tx0 + 1.0)
        area_pen = jnp.maximum(pred_area - true_area, 0.0) / (true_area + 1.0)
        cdy = (py0 + py1) / 2.0 - (ty0 + ty1) / 2.0
        cdx = (px0 + px1) / 2.0 - (tx0 + tx1) / 2.0
        center_pen = _sqrt16(cdy * cdy + cdx * cdx) / 20.0
        iy0 = jnp.maximum(py0, ty0)
        ix0 = jnp.maximum(px0, tx0)
        iy1 = jnp.minimum(py1, ty1)
        ix1 = jnp.minimum(px1, tx1)
        inter = (jnp.maximum(0.0, iy1 - iy0 + 1.0)
                 * jnp.maximum(0.0, ix1 - ix0 + 1.0))
        union = pred_area + true_area - inter + 1e-6
        total = area_pen + center_pen + 1.0 - inter / union
        pen = jnp.where(p_any & t_any, _tanh16(total), _splat(0.0))
        res_v[...] = pen
        pltpu.sync_copy(res_v, out_hbm.at[b])


def kernel(prediction_probs, expected_onehot):
    pred_t = jnp.transpose(prediction_probs, (0, 1, 3, 2))
    true_t = jnp.transpose(expected_onehot, (0, 1, 3, 2))
    p_ymin, p_ymax, t_ymin, t_ymax = _bbox_vectors(pred_t, true_t)
    pens = _make_penalty_kernel()(p_ymin, p_ymax, t_ymin, t_ymax)
    return 0.05 * jnp.mean(pens[:, 0])
